# Initial kernel scaffold; baseline (speedup 1.0000x reference)
import jax
import jax.numpy as jnp
from jax.experimental import pallas as pl


def kernel(x, edge_index, W_self1, W_neigh1, b1, W_self2, W_neigh2, b2):
    raise NotImplementedError("write your pallas kernel here")


# speedup vs baseline: 5.0183x; 5.0183x over previous
import functools

import jax
import jax.numpy as jnp
from jax import lax
from jax.experimental import pallas as pl
from jax.experimental.pallas import tpu as pltpu
from jax.experimental.pallas import tpu_sc as plsc

N = 10000
E = 320000
D = 128
NC = 2
NS = 16
NW = NC * NS
EPW = E // NW
CHUNK = 80
NCHUNK = EPW // CHUNK
RPT = 624
REM = N - NS * RPT

_MESH = plsc.VectorSubcoreMesh(
    core_axis_name="c", subcore_axis_name="s", num_cores=NC, num_subcores=NS
)


_ZCHUNKS = [(k * CHUNK, CHUNK) for k in range(7)] + [(7 * CHUNK, 64)]


def _tile_rows_io(sid, spmem_of, buf, hbm_of, to_hbm):
    for off, sz in _ZCHUNKS:
        r = pl.ds(sid * RPT + off, sz)
        b = pl.ds(0, sz)
        if to_hbm:
            pltpu.sync_copy(spmem_of(r), buf.at[b])
            pltpu.sync_copy(buf.at[b], hbm_of(r))
        else:
            pltpu.sync_copy(hbm_of(r), buf.at[b])
            pltpu.sync_copy(buf.at[b], spmem_of(r))

    @pl.when(sid == 0)
    def _():
        r = pl.ds(NS * RPT, REM)
        b = pl.ds(0, REM)
        if to_hbm:
            pltpu.sync_copy(spmem_of(r), buf.at[b])
            pltpu.sync_copy(buf.at[b], hbm_of(r))
        else:
            pltpu.sync_copy(hbm_of(r), buf.at[b])
            pltpu.sync_copy(buf.at[b], spmem_of(r))


def _tile_zero_spmem(sid, spmem_of, buf):
    for off, sz in _ZCHUNKS:
        pltpu.sync_copy(buf.at[pl.ds(0, sz)], spmem_of(pl.ds(sid * RPT + off, sz)))

    @pl.when(sid == 0)
    def _():
        pltpu.sync_copy(buf.at[pl.ds(0, REM)], spmem_of(pl.ds(NS * RPT, REM)))


def _seg_sum(table, srcr, dstr, zerosD, with_deg):
    out_type = [jax.ShapeDtypeStruct((NC, N, D), jnp.float32)]
    scratch = [
        pltpu.VMEM((CHUNK,), jnp.int32),
        pltpu.VMEM((CHUNK,), jnp.int32),
        pltpu.VMEM((CHUNK, D), jnp.float32),
        pltpu.VMEM_SHARED((N, D), jnp.float32),
    ]
    if with_deg:
        out_type.append(jax.ShapeDtypeStruct((NW * N,), jnp.float32))
        scratch.append(pltpu.VMEM((N,), jnp.float32))

    @functools.partial(
        pl.kernel, out_type=tuple(out_type), mesh=_MESH, scratch_types=scratch,
        compiler_params=pltpu.CompilerParams(needs_layout_passes=False),
    )
    def k(table_h, src_h, dst_h, zd_h, out_h, *rest):
        if with_deg:
            deg_h, srcv, dstv, rows, accS, hist = rest
        else:
            srcv, dstv, rows, accS = rest
        cid = lax.axis_index("c")
        sid = lax.axis_index("s")
        wid = sid * NC + cid
        base = wid * EPW
        pltpu.sync_copy(zd_h, rows)
        _tile_zero_spmem(sid, lambda s: accS.at[s], rows)
        if with_deg:
            z16 = jnp.zeros((16,), jnp.float32)

            def zstep(i, carry):
                hist[pl.ds(i * 16, 16)] = z16
                return carry

            lax.fori_loop(0, N // 16, zstep, 0)
        plsc.subcore_barrier()
        ones16 = jnp.ones((16,), jnp.float32)

        def step(j, carry):
            pltpu.sync_copy(src_h.at[pl.ds(base + j * CHUNK, CHUNK)], srcv)
            pltpu.sync_copy(dst_h.at[pl.ds(base + j * CHUNK, CHUNK)], dstv)
            pltpu.sync_copy(table_h.at[srcv], rows)
            pltpu.sync_copy(rows, accS.at[dstv], add=True)
            if with_deg:
                for kk in range(CHUNK // 16):
                    idx = dstv[pl.ds(kk * 16, 16)]
                    plsc.addupdate_scatter(hist, [idx], ones16)
            return carry

        lax.fori_loop(0, NCHUNK, step, 0)
        plsc.subcore_barrier()
        _tile_rows_io(sid, lambda s: accS.at[s], rows,
                      lambda s: out_h.at[cid, s], to_hbm=True)
        if with_deg:
            pltpu.sync_copy(hist, deg_h.at[pl.ds(wid * N, N)])

    return k(table, srcr, dstr, zerosD)


_ROWBLK = 1000
_GRID = (N // _ROWBLK,)


def _tc_pre(h, Wn, Ws, b):
    def body(h_ref, wn_ref, ws_ref, b_ref, p_ref, s_ref):
        hb = h_ref[...]
        p_ref[...] = jnp.dot(hb, wn_ref[...], preferred_element_type=jnp.float32)
        s_ref[...] = jnp.dot(hb, ws_ref[...], preferred_element_type=jnp.float32) + b_ref[...]

    return pl.pallas_call(
        body,
        grid=_GRID,
        in_specs=[
            pl.BlockSpec((_ROWBLK, D), lambda i: (i, 0)),
            pl.BlockSpec((D, D), lambda i: (0, 0)),
            pl.BlockSpec((D, D), lambda i: (0, 0)),
            pl.BlockSpec((1, D), lambda i: (0, 0)),
        ],
        out_specs=[
            pl.BlockSpec((_ROWBLK, D), lambda i: (i, 0)),
            pl.BlockSpec((_ROWBLK, D), lambda i: (i, 0)),
        ],
        out_shape=[
            jax.ShapeDtypeStruct((N, D), jnp.float32),
            jax.ShapeDtypeStruct((N, D), jnp.float32),
        ],
    )(h, Wn, Ws, b.reshape(1, D))


def _tc_mid(s1, agg1, deg, Wn, Ws, b):
    def body(s_ref, a_ref, d_ref, wn_ref, ws_ref, b_ref, p_ref, s2_ref):
        a = a_ref[0] + a_ref[1]
        degc = jnp.sum(d_ref[...], axis=1, keepdims=True)
        h1 = s_ref[...] + a / jnp.maximum(degc, 1.0)
        p_ref[...] = jnp.dot(h1, wn_ref[...], preferred_element_type=jnp.float32)
        s2_ref[...] = jnp.dot(h1, ws_ref[...], preferred_element_type=jnp.float32) + b_ref[...]

    return pl.pallas_call(
        body,
        grid=_GRID,
        in_specs=[
            pl.BlockSpec((_ROWBLK, D), lambda i: (i, 0)),
            pl.BlockSpec((NC, _ROWBLK, D), lambda i: (0, i, 0)),
            pl.BlockSpec((_ROWBLK, NW), lambda i: (i, 0)),
            pl.BlockSpec((D, D), lambda i: (0, 0)),
            pl.BlockSpec((D, D), lambda i: (0, 0)),
            pl.BlockSpec((1, D), lambda i: (0, 0)),
        ],
        out_specs=[
            pl.BlockSpec((_ROWBLK, D), lambda i: (i, 0)),
            pl.BlockSpec((_ROWBLK, D), lambda i: (i, 0)),
        ],
        out_shape=[
            jax.ShapeDtypeStruct((N, D), jnp.float32),
            jax.ShapeDtypeStruct((N, D), jnp.float32),
        ],
    )(s1, agg1, deg, Wn, Ws, b.reshape(1, D))


def _tc_post(s2, agg2, deg):
    def body(s_ref, a_ref, d_ref, o_ref):
        a = a_ref[0] + a_ref[1]
        degc = jnp.sum(d_ref[...], axis=1, keepdims=True)
        h2 = s_ref[...] + a / jnp.maximum(degc, 1.0)
        norm = jnp.sqrt(jnp.sum(h2 * h2, axis=1, keepdims=True))
        o_ref[...] = h2 / jnp.maximum(norm, 1e-12)

    return pl.pallas_call(
        body,
        grid=_GRID,
        in_specs=[
            pl.BlockSpec((_ROWBLK, D), lambda i: (i, 0)),
            pl.BlockSpec((NC, _ROWBLK, D), lambda i: (0, i, 0)),
            pl.BlockSpec((_ROWBLK, NW), lambda i: (i, 0)),
        ],
        out_specs=pl.BlockSpec((_ROWBLK, D), lambda i: (i, 0)),
        out_shape=jax.ShapeDtypeStruct((N, D), jnp.float32),
    )(s2, agg2, deg)


def kernel(x, edge_index, W_self1, W_neigh1, b1, W_self2, W_neigh2, b2):
    src = edge_index[0].astype(jnp.int32)
    dst = edge_index[1].astype(jnp.int32)
    zerosD = jnp.zeros((CHUNK, D), jnp.float32)

    p1, s1 = _tc_pre(x, W_neigh1, W_self1, b1)
    agg1, deg_flat = _seg_sum(p1, src, dst, zerosD, with_deg=True)
    degT = deg_flat.reshape(NW, N).T
    p2, s2 = _tc_mid(s1, agg1, degT, W_neigh2, W_self2, b2)
    agg2, = _seg_sum(p2, src, dst, zerosD, with_deg=False)
    return _tc_post(s2, agg2, degT)


# speedup vs baseline: 8.7965x; 1.7529x over previous
import functools

import jax
import jax.numpy as jnp
from jax import lax
from jax.experimental import pallas as pl
from jax.experimental.pallas import tpu as pltpu
from jax.experimental.pallas import tpu_sc as plsc

N = 10000
E = 320000
D = 128
NC = 2
NS = 16
NW = NC * NS
EPW = E // NW
CHUNK = 80
NCHUNK = EPW // CHUNK
RPT = 624
REM = N - NS * RPT

_MESH = plsc.VectorSubcoreMesh(
    core_axis_name="c", subcore_axis_name="s", num_cores=NC, num_subcores=NS
)


_ZCHUNKS = [(k * CHUNK, CHUNK) for k in range(7)] + [(7 * CHUNK, 64)]


NPAIR = NCHUNK // 2


def _seg_sum(table, ei, zerosD, with_deg):
    out_type = [jax.ShapeDtypeStruct((NC, N, D), jnp.float32)]
    scratch = [
        pltpu.VMEM((2, 2, CHUNK), jnp.int32),
        pltpu.VMEM((2, 2, CHUNK), jnp.int32),
        pltpu.VMEM((CHUNK, D), jnp.float32),
        pltpu.VMEM((CHUNK, D), jnp.float32),
        pltpu.VMEM_SHARED((N, D), jnp.float32),
        pltpu.SemaphoreType.DMA,
        pltpu.SemaphoreType.DMA,
        pltpu.SemaphoreType.DMA,
        pltpu.SemaphoreType.DMA,
        pltpu.SemaphoreType.DMA,
        pltpu.SemaphoreType.DMA,
    ]
    if with_deg:
        out_type.append(jax.ShapeDtypeStruct((NW * N,), jnp.float32))
        scratch.append(pltpu.VMEM((N,), jnp.float32))

    @functools.partial(
        pl.kernel, out_type=tuple(out_type), mesh=_MESH, scratch_types=scratch,
        compiler_params=pltpu.CompilerParams(needs_layout_passes=False),
    )
    def k(table_h, ei_h, zd_h, out_h, *rest):
        if with_deg:
            deg_h, idxpA, idxpB, rows0, rows1, accS, iA, iB, g0, g1, s0, s1, hist = rest
        else:
            idxpA, idxpB, rows0, rows1, accS, iA, iB, g0, g1, s0, s1 = rest
        cid = lax.axis_index("c")
        sid = lax.axis_index("s")
        wid = sid * NC + cid
        ones16 = jnp.ones((16,), jnp.float32)

        def hist_update(idxp, a):
            if with_deg:
                for kk in range(CHUNK // 16):
                    idx = idxp[a, 1, pl.ds(kk * 16, 16)]
                    plsc.addupdate_scatter(hist, [idx], ones16)

        pltpu.sync_copy(zd_h, rows0)
        for off, sz in _ZCHUNKS:
            pltpu.async_copy(rows0.at[pl.ds(0, sz)],
                             accS.at[pl.ds(sid * RPT + off, sz)], g0)

        @pl.when(sid == 0)
        def _():
            pltpu.async_copy(rows0.at[pl.ds(0, REM)],
                             accS.at[pl.ds(NS * RPT, REM)], g0)

        if with_deg:
            z16 = jnp.zeros((16,), jnp.float32)

            def zstep(i, carry):
                hist[pl.ds(i * 16, 16)] = z16
                return carry

            lax.fori_loop(0, N // 16, zstep, 0)
        for off, sz in _ZCHUNKS:
            pltpu.make_async_copy(rows0.at[pl.ds(0, sz)],
                                  accS.at[pl.ds(sid * RPT + off, sz)], g0).wait()

        @pl.when(sid == 0)
        def _():
            pltpu.make_async_copy(rows0.at[pl.ds(0, REM)],
                                  accS.at[pl.ds(NS * RPT, REM)], g0).wait()

        plsc.subcore_barrier()

        def wait_idx(u, idxp, isem):
            pltpu.make_async_copy(ei_h.at[wid, pl.ds(2 * u, 2)], idxp, isem).wait()

        def load_idx(u, idxp, isem):
            pltpu.async_copy(ei_h.at[wid, pl.ds(2 * u, 2)], idxp, isem)

        def wait_scat(rows, idxp, a, ssem):
            pltpu.make_async_copy(rows, accS.at[idxp.at[a, 1]], ssem).wait()

        def do_pair(u, idxp, isem, wait_prev_scat, load_next):
            wait_idx(u, idxp, isem)
            if wait_prev_scat:
                wait_scat(rows0, idxp, 0, s0)
                wait_scat(rows1, idxp, 1, s1)
            if load_next is not None:
                load_next()
            pltpu.async_copy(table_h.at[idxp.at[0, 0]], rows0, g0)
            pltpu.async_copy(table_h.at[idxp.at[1, 0]], rows1, g1)
            hist_update(idxp, 0)
            hist_update(idxp, 1)
            pltpu.make_async_copy(table_h.at[idxp.at[0, 0]], rows0, g0).wait()
            pltpu.async_copy(rows0, accS.at[idxp.at[0, 1]], s0, add=True)
            pltpu.make_async_copy(table_h.at[idxp.at[1, 0]], rows1, g1).wait()
            pltpu.async_copy(rows1, accS.at[idxp.at[1, 1]], s1, add=True)

        pltpu.sync_copy(ei_h.at[wid, pl.ds(0, 2)], idxpA)
        load_idx(1, idxpB, iB)
        pltpu.async_copy(table_h.at[idxpA.at[0, 0]], rows0, g0)
        pltpu.async_copy(table_h.at[idxpA.at[1, 0]], rows1, g1)
        hist_update(idxpA, 0)
        hist_update(idxpA, 1)
        pltpu.make_async_copy(table_h.at[idxpA.at[0, 0]], rows0, g0).wait()
        pltpu.async_copy(rows0, accS.at[idxpA.at[0, 1]], s0, add=True)
        pltpu.make_async_copy(table_h.at[idxpA.at[1, 0]], rows1, g1).wait()
        pltpu.async_copy(rows1, accS.at[idxpA.at[1, 1]], s1, add=True)

        def super_body(ss, carry):
            u = 2 * ss + 1
            do_pair(u, idxpB, iB, True,
                    lambda: load_idx(u + 1, idxpA, iA))
            do_pair(u + 1, idxpA, iA, True,
                    lambda: load_idx(u + 2, idxpB, iB))
            return carry

        lax.fori_loop(0, (NPAIR - 2) // 2, super_body, 0)

        def load_tail():
            pltpu.async_copy(ei_h.at[wid, pl.ds(NCHUNK - 1, 1)],
                             idxpA.at[pl.ds(0, 1)], iA)

        do_pair(NPAIR - 1, idxpB, iB, True, load_tail)
        pltpu.make_async_copy(ei_h.at[wid, pl.ds(NCHUNK - 1, 1)],
                              idxpA.at[pl.ds(0, 1)], iA).wait()
        wait_scat(rows0, idxpA, 0, s0)
        pltpu.async_copy(table_h.at[idxpA.at[0, 0]], rows0, g0)
        hist_update(idxpA, 0)
        pltpu.make_async_copy(table_h.at[idxpA.at[0, 0]], rows0, g0).wait()
        pltpu.async_copy(rows0, accS.at[idxpA.at[0, 1]], s0, add=True)
        wait_scat(rows0, idxpA, 0, s0)
        wait_scat(rows1, idxpB, 1, s1)

        plsc.subcore_barrier()
        bufs = (rows0, rows1)
        sems = (g0, g1)
        for r, (off, sz) in enumerate(_ZCHUNKS):
            p = r % 2
            if r >= 2:
                po, psz = _ZCHUNKS[r - 2]
                pltpu.make_async_copy(
                    bufs[p].at[pl.ds(0, psz)],
                    out_h.at[cid, pl.ds(sid * RPT + po, psz)], sems[p]).wait()
            pltpu.sync_copy(accS.at[pl.ds(sid * RPT + off, sz)],
                            bufs[p].at[pl.ds(0, sz)])
            pltpu.async_copy(bufs[p].at[pl.ds(0, sz)],
                             out_h.at[cid, pl.ds(sid * RPT + off, sz)], sems[p])
        for r in (len(_ZCHUNKS) - 2, len(_ZCHUNKS) - 1):
            off, sz = _ZCHUNKS[r]
            pltpu.make_async_copy(
                bufs[r % 2].at[pl.ds(0, sz)],
                out_h.at[cid, pl.ds(sid * RPT + off, sz)], sems[r % 2]).wait()

        @pl.when(sid == 0)
        def _():
            pltpu.sync_copy(accS.at[pl.ds(NS * RPT, REM)], rows0.at[pl.ds(0, REM)])
            pltpu.sync_copy(rows0.at[pl.ds(0, REM)],
                            out_h.at[cid, pl.ds(NS * RPT, REM)])

        if with_deg:
            pltpu.sync_copy(hist, deg_h.at[pl.ds(wid * N, N)])

    return k(table, ei, zerosD)


_ROWBLK = 1000
_GRID = (N // _ROWBLK,)


def _tc_pre(h, Wn, Ws, b):
    def body(h_ref, wn_ref, ws_ref, b_ref, p_ref, s_ref):
        hb = h_ref[...]
        p_ref[...] = jnp.dot(hb, wn_ref[...], preferred_element_type=jnp.float32)
        s_ref[...] = jnp.dot(hb, ws_ref[...], preferred_element_type=jnp.float32) + b_ref[...]

    return pl.pallas_call(
        body,
        grid=_GRID,
        in_specs=[
            pl.BlockSpec((_ROWBLK, D), lambda i: (i, 0)),
            pl.BlockSpec((D, D), lambda i: (0, 0)),
            pl.BlockSpec((D, D), lambda i: (0, 0)),
            pl.BlockSpec((1, D), lambda i: (0, 0)),
        ],
        out_specs=[
            pl.BlockSpec((_ROWBLK, D), lambda i: (i, 0)),
            pl.BlockSpec((_ROWBLK, D), lambda i: (i, 0)),
        ],
        out_shape=[
            jax.ShapeDtypeStruct((N, D), jnp.float32),
            jax.ShapeDtypeStruct((N, D), jnp.float32),
        ],
    )(h, Wn, Ws, b.reshape(1, D))


def _tc_mid(s1, agg1, deg, Wn, Ws, b):
    def body(s_ref, a_ref, d_ref, wn_ref, ws_ref, b_ref, p_ref, s2_ref):
        a = a_ref[0] + a_ref[1]
        degc = jnp.sum(d_ref[...], axis=1, keepdims=True)
        h1 = s_ref[...] + a / jnp.maximum(degc, 1.0)
        p_ref[...] = jnp.dot(h1, wn_ref[...], preferred_element_type=jnp.float32)
        s2_ref[...] = jnp.dot(h1, ws_ref[...], preferred_element_type=jnp.float32) + b_ref[...]

    return pl.pallas_call(
        body,
        grid=_GRID,
        in_specs=[
            pl.BlockSpec((_ROWBLK, D), lambda i: (i, 0)),
            pl.BlockSpec((NC, _ROWBLK, D), lambda i: (0, i, 0)),
            pl.BlockSpec((_ROWBLK, NW), lambda i: (i, 0)),
            pl.BlockSpec((D, D), lambda i: (0, 0)),
            pl.BlockSpec((D, D), lambda i: (0, 0)),
            pl.BlockSpec((1, D), lambda i: (0, 0)),
        ],
        out_specs=[
            pl.BlockSpec((_ROWBLK, D), lambda i: (i, 0)),
            pl.BlockSpec((_ROWBLK, D), lambda i: (i, 0)),
        ],
        out_shape=[
            jax.ShapeDtypeStruct((N, D), jnp.float32),
            jax.ShapeDtypeStruct((N, D), jnp.float32),
        ],
    )(s1, agg1, deg, Wn, Ws, b.reshape(1, D))


def _tc_post(s2, agg2, deg):
    def body(s_ref, a_ref, d_ref, o_ref):
        a = a_ref[0] + a_ref[1]
        degc = jnp.sum(d_ref[...], axis=1, keepdims=True)
        h2 = s_ref[...] + a / jnp.maximum(degc, 1.0)
        norm = jnp.sqrt(jnp.sum(h2 * h2, axis=1, keepdims=True))
        o_ref[...] = h2 / jnp.maximum(norm, 1e-12)

    return pl.pallas_call(
        body,
        grid=_GRID,
        in_specs=[
            pl.BlockSpec((_ROWBLK, D), lambda i: (i, 0)),
            pl.BlockSpec((NC, _ROWBLK, D), lambda i: (0, i, 0)),
            pl.BlockSpec((_ROWBLK, NW), lambda i: (i, 0)),
        ],
        out_specs=pl.BlockSpec((_ROWBLK, D), lambda i: (i, 0)),
        out_shape=jax.ShapeDtypeStruct((N, D), jnp.float32),
    )(s2, agg2, deg)


def kernel(x, edge_index, W_self1, W_neigh1, b1, W_self2, W_neigh2, b2):
    src = edge_index[0].astype(jnp.int32)
    dst = edge_index[1].astype(jnp.int32)
    ei = jnp.stack(
        (src.reshape(NW, NCHUNK, CHUNK), dst.reshape(NW, NCHUNK, CHUNK)),
        axis=2)
    zerosD = jnp.zeros((CHUNK, D), jnp.float32)

    p1, s1 = _tc_pre(x, W_neigh1, W_self1, b1)
    agg1, deg_flat = _seg_sum(p1, ei, zerosD, with_deg=True)
    degT = deg_flat.reshape(NW, N).T
    p2, s2 = _tc_mid(s1, agg1, degT, W_neigh2, W_self2, b2)
    agg2, = _seg_sum(p2, ei, zerosD, with_deg=False)
    return _tc_post(s2, agg2, degT)


# speedup vs baseline: 9.1100x; 1.0356x over previous
import functools

import jax
import jax.numpy as jnp
from jax import lax
from jax.experimental import pallas as pl
from jax.experimental.pallas import tpu as pltpu
from jax.experimental.pallas import tpu_sc as plsc

N = 10000
E = 320000
D = 128
NC = 2
NS = 16
NW = NC * NS
EPW = E // NW
CHUNK = 80
NCHUNK = EPW // CHUNK
RPT = 624
REM = N - NS * RPT

_MESH = plsc.VectorSubcoreMesh(
    core_axis_name="c", subcore_axis_name="s", num_cores=NC, num_subcores=NS
)


_ZCHUNKS = [(k * CHUNK, CHUNK) for k in range(7)] + [(7 * CHUNK, 64)]


NPAIR = NCHUNK // 2


def _seg_sum(table, src1, dst1, zerosD, with_deg):
    out_type = [jax.ShapeDtypeStruct((NC, N, D), jnp.float32)]
    scratch = [
        pltpu.VMEM((2, 2, CHUNK), jnp.int32),
        pltpu.VMEM((2, 2, CHUNK), jnp.int32),
        pltpu.VMEM((CHUNK, D), jnp.float32),
        pltpu.VMEM((CHUNK, D), jnp.float32),
        pltpu.VMEM_SHARED((N, D), jnp.float32),
        pltpu.SemaphoreType.DMA,
        pltpu.SemaphoreType.DMA,
        pltpu.SemaphoreType.DMA,
        pltpu.SemaphoreType.DMA,
        pltpu.SemaphoreType.DMA,
        pltpu.SemaphoreType.DMA,
    ]
    if with_deg:
        out_type.append(jax.ShapeDtypeStruct((NW * N,), jnp.float32))
        scratch.append(pltpu.VMEM((N,), jnp.float32))

    @functools.partial(
        pl.kernel, out_type=tuple(out_type), mesh=_MESH, scratch_types=scratch,
        compiler_params=pltpu.CompilerParams(needs_layout_passes=False),
    )
    def k(table_h, src_h, dst_h, zd_h, out_h, *rest):
        if with_deg:
            deg_h, idxpA, idxpB, rows0, rows1, accS, iA, iB, g0, g1, s0, s1, hist = rest
        else:
            idxpA, idxpB, rows0, rows1, accS, iA, iB, g0, g1, s0, s1 = rest
        cid = lax.axis_index("c")
        sid = lax.axis_index("s")
        wid = sid * NC + cid
        base = wid * EPW
        ones16 = jnp.ones((16,), jnp.float32)

        def hist_update(idxp, a):
            if with_deg:
                for kk in range(CHUNK // 16):
                    idx = idxp[a, 1, pl.ds(kk * 16, 16)]
                    plsc.addupdate_scatter(hist, [idx], ones16)

        pltpu.sync_copy(zd_h, rows0)
        for off, sz in _ZCHUNKS:
            pltpu.async_copy(rows0.at[pl.ds(0, sz)],
                             accS.at[pl.ds(sid * RPT + off, sz)], g0)

        @pl.when(sid == 0)
        def _():
            pltpu.async_copy(rows0.at[pl.ds(0, REM)],
                             accS.at[pl.ds(NS * RPT, REM)], g0)

        if with_deg:
            z16 = jnp.zeros((16,), jnp.float32)

            def zstep(i, carry):
                hist[pl.ds(i * 16, 16)] = z16
                return carry

            lax.fori_loop(0, N // 16, zstep, 0)
        for off, sz in _ZCHUNKS:
            pltpu.make_async_copy(rows0.at[pl.ds(0, sz)],
                                  accS.at[pl.ds(sid * RPT + off, sz)], g0).wait()

        @pl.when(sid == 0)
        def _():
            pltpu.make_async_copy(rows0.at[pl.ds(0, REM)],
                                  accS.at[pl.ds(NS * RPT, REM)], g0).wait()

        plsc.subcore_barrier()

        def idx_copies(u, idxp):
            c0 = base + 2 * u * CHUNK
            return [(src_h.at[pl.ds(c0, CHUNK)], idxp.at[0, 0]),
                    (dst_h.at[pl.ds(c0, CHUNK)], idxp.at[0, 1]),
                    (src_h.at[pl.ds(c0 + CHUNK, CHUNK)], idxp.at[1, 0]),
                    (dst_h.at[pl.ds(c0 + CHUNK, CHUNK)], idxp.at[1, 1])]

        def wait_idx(u, idxp, isem):
            for s_, d_ in idx_copies(u, idxp):
                pltpu.make_async_copy(s_, d_, isem).wait()

        def load_idx(u, idxp, isem):
            for s_, d_ in idx_copies(u, idxp):
                pltpu.async_copy(s_, d_, isem)

        def wait_scat(rows, idxp, a, ssem):
            pltpu.make_async_copy(rows, accS.at[idxp.at[a, 1]], ssem).wait()

        def do_pair(u, idxp, isem, wait_prev_scat, load_next):
            wait_idx(u, idxp, isem)
            if wait_prev_scat:
                wait_scat(rows0, idxp, 0, s0)
                wait_scat(rows1, idxp, 1, s1)
            if load_next is not None:
                load_next()
            pltpu.async_copy(table_h.at[idxp.at[0, 0]], rows0, g0)
            pltpu.async_copy(table_h.at[idxp.at[1, 0]], rows1, g1)
            hist_update(idxp, 0)
            hist_update(idxp, 1)
            pltpu.make_async_copy(table_h.at[idxp.at[0, 0]], rows0, g0).wait()
            pltpu.async_copy(rows0, accS.at[idxp.at[0, 1]], s0, add=True)
            pltpu.make_async_copy(table_h.at[idxp.at[1, 0]], rows1, g1).wait()
            pltpu.async_copy(rows1, accS.at[idxp.at[1, 1]], s1, add=True)

        for s_, d_ in idx_copies(0, idxpA):
            pltpu.sync_copy(s_, d_)
        load_idx(1, idxpB, iB)
        pltpu.async_copy(table_h.at[idxpA.at[0, 0]], rows0, g0)
        pltpu.async_copy(table_h.at[idxpA.at[1, 0]], rows1, g1)
        hist_update(idxpA, 0)
        hist_update(idxpA, 1)
        pltpu.make_async_copy(table_h.at[idxpA.at[0, 0]], rows0, g0).wait()
        pltpu.async_copy(rows0, accS.at[idxpA.at[0, 1]], s0, add=True)
        pltpu.make_async_copy(table_h.at[idxpA.at[1, 0]], rows1, g1).wait()
        pltpu.async_copy(rows1, accS.at[idxpA.at[1, 1]], s1, add=True)

        def super_body(ss, carry):
            u = 2 * ss + 1
            do_pair(u, idxpB, iB, True,
                    lambda: load_idx(u + 1, idxpA, iA))
            do_pair(u + 1, idxpA, iA, True,
                    lambda: load_idx(u + 2, idxpB, iB))
            return carry

        lax.fori_loop(0, (NPAIR - 2) // 2, super_body, 0)

        ct = base + (NCHUNK - 1) * CHUNK
        tail_copies = [(src_h.at[pl.ds(ct, CHUNK)], idxpA.at[0, 0]),
                       (dst_h.at[pl.ds(ct, CHUNK)], idxpA.at[0, 1])]

        def load_tail():
            for s_, d_ in tail_copies:
                pltpu.async_copy(s_, d_, iA)

        do_pair(NPAIR - 1, idxpB, iB, True, load_tail)
        for s_, d_ in tail_copies:
            pltpu.make_async_copy(s_, d_, iA).wait()
        wait_scat(rows0, idxpA, 0, s0)
        pltpu.async_copy(table_h.at[idxpA.at[0, 0]], rows0, g0)
        hist_update(idxpA, 0)
        pltpu.make_async_copy(table_h.at[idxpA.at[0, 0]], rows0, g0).wait()
        pltpu.async_copy(rows0, accS.at[idxpA.at[0, 1]], s0, add=True)
        wait_scat(rows0, idxpA, 0, s0)
        wait_scat(rows1, idxpB, 1, s1)

        plsc.subcore_barrier()
        bufs = (rows0, rows1)
        sems = (g0, g1)
        for r, (off, sz) in enumerate(_ZCHUNKS):
            p = r % 2
            if r >= 2:
                po, psz = _ZCHUNKS[r - 2]
                pltpu.make_async_copy(
                    bufs[p].at[pl.ds(0, psz)],
                    out_h.at[cid, pl.ds(sid * RPT + po, psz)], sems[p]).wait()
            pltpu.sync_copy(accS.at[pl.ds(sid * RPT + off, sz)],
                            bufs[p].at[pl.ds(0, sz)])
            pltpu.async_copy(bufs[p].at[pl.ds(0, sz)],
                             out_h.at[cid, pl.ds(sid * RPT + off, sz)], sems[p])
        for r in (len(_ZCHUNKS) - 2, len(_ZCHUNKS) - 1):
            off, sz = _ZCHUNKS[r]
            pltpu.make_async_copy(
                bufs[r % 2].at[pl.ds(0, sz)],
                out_h.at[cid, pl.ds(sid * RPT + off, sz)], sems[r % 2]).wait()

        @pl.when(sid == 0)
        def _():
            pltpu.sync_copy(accS.at[pl.ds(NS * RPT, REM)], rows0.at[pl.ds(0, REM)])
            pltpu.sync_copy(rows0.at[pl.ds(0, REM)],
                            out_h.at[cid, pl.ds(NS * RPT, REM)])

        if with_deg:
            pltpu.sync_copy(hist, deg_h.at[pl.ds(wid * N, N)])

    return k(table, src1, dst1, zerosD)


_ROWBLK = 1000
_GRID = (N // _ROWBLK,)


def _tc_pre(h, Wn, Ws, b):
    def body(h_ref, wn_ref, ws_ref, b_ref, p_ref, s_ref):
        hb = h_ref[...]
        p_ref[...] = jnp.dot(hb, wn_ref[...], preferred_element_type=jnp.float32)
        s_ref[...] = jnp.dot(hb, ws_ref[...], preferred_element_type=jnp.float32) + b_ref[...]

    return pl.pallas_call(
        body,
        grid=_GRID,
        in_specs=[
            pl.BlockSpec((_ROWBLK, D), lambda i: (i, 0)),
            pl.BlockSpec((D, D), lambda i: (0, 0)),
            pl.BlockSpec((D, D), lambda i: (0, 0)),
            pl.BlockSpec((1, D), lambda i: (0, 0)),
        ],
        out_specs=[
            pl.BlockSpec((_ROWBLK, D), lambda i: (i, 0)),
            pl.BlockSpec((_ROWBLK, D), lambda i: (i, 0)),
        ],
        out_shape=[
            jax.ShapeDtypeStruct((N, D), jnp.float32),
            jax.ShapeDtypeStruct((N, D), jnp.float32),
        ],
    )(h, Wn, Ws, b.reshape(1, D))


def _tc_mid(s1, agg1, deg, Wn, Ws, b):
    def body(s_ref, a_ref, d_ref, wn_ref, ws_ref, b_ref, p_ref, s2_ref):
        a = a_ref[0] + a_ref[1]
        degc = jnp.sum(d_ref[...], axis=1, keepdims=True)
        h1 = s_ref[...] + a / jnp.maximum(degc, 1.0)
        p_ref[...] = jnp.dot(h1, wn_ref[...], preferred_element_type=jnp.float32)
        s2_ref[...] = jnp.dot(h1, ws_ref[...], preferred_element_type=jnp.float32) + b_ref[...]

    return pl.pallas_call(
        body,
        grid=_GRID,
        in_specs=[
            pl.BlockSpec((_ROWBLK, D), lambda i: (i, 0)),
            pl.BlockSpec((NC, _ROWBLK, D), lambda i: (0, i, 0)),
            pl.BlockSpec((_ROWBLK, NW), lambda i: (i, 0)),
            pl.BlockSpec((D, D), lambda i: (0, 0)),
            pl.BlockSpec((D, D), lambda i: (0, 0)),
            pl.BlockSpec((1, D), lambda i: (0, 0)),
        ],
        out_specs=[
            pl.BlockSpec((_ROWBLK, D), lambda i: (i, 0)),
            pl.BlockSpec((_ROWBLK, D), lambda i: (i, 0)),
        ],
        out_shape=[
            jax.ShapeDtypeStruct((N, D), jnp.float32),
            jax.ShapeDtypeStruct((N, D), jnp.float32),
        ],
    )(s1, agg1, deg, Wn, Ws, b.reshape(1, D))


def _tc_post(s2, agg2, deg):
    def body(s_ref, a_ref, d_ref, o_ref):
        a = a_ref[0] + a_ref[1]
        degc = jnp.sum(d_ref[...], axis=1, keepdims=True)
        h2 = s_ref[...] + a / jnp.maximum(degc, 1.0)
        norm = jnp.sqrt(jnp.sum(h2 * h2, axis=1, keepdims=True))
        o_ref[...] = h2 / jnp.maximum(norm, 1e-12)

    return pl.pallas_call(
        body,
        grid=_GRID,
        in_specs=[
            pl.BlockSpec((_ROWBLK, D), lambda i: (i, 0)),
            pl.BlockSpec((NC, _ROWBLK, D), lambda i: (0, i, 0)),
            pl.BlockSpec((_ROWBLK, NW), lambda i: (i, 0)),
        ],
        out_specs=pl.BlockSpec((_ROWBLK, D), lambda i: (i, 0)),
        out_shape=jax.ShapeDtypeStruct((N, D), jnp.float32),
    )(s2, agg2, deg)


def kernel(x, edge_index, W_self1, W_neigh1, b1, W_self2, W_neigh2, b2):
    src = edge_index[0].astype(jnp.int32)
    dst = edge_index[1].astype(jnp.int32)
    zerosD = jnp.zeros((CHUNK, D), jnp.float32)

    p1, s1 = _tc_pre(x, W_neigh1, W_self1, b1)
    agg1, deg_flat = _seg_sum(p1, src, dst, zerosD, with_deg=True)
    degT = deg_flat.reshape(NW, N).T
    p2, s2 = _tc_mid(s1, agg1, degT, W_neigh2, W_self2, b2)
    agg2, = _seg_sum(p2, src, dst, zerosD, with_deg=False)
    return _tc_post(s2, agg2, degT)
